# Initial kernel scaffold; baseline (speedup 1.0000x reference)
import jax
import jax.numpy as jnp
from jax.experimental import pallas as pl


def kernel(center_word, context_word, neg_words, center_embeddings, context_embeddings):
    raise NotImplementedError("write your pallas kernel here")


# speedup vs baseline: 5.2361x; 5.2361x over previous
import functools

import jax
import jax.numpy as jnp
from jax import lax
from jax.experimental import pallas as pl
from jax.experimental.pallas import tpu as pltpu
from jax.experimental.pallas import tpu_sc as plsc

VOCAB = 1000000
DIM = 64
B = 16384
NEG = 20

NC = 2
NS = 16
NW = NC * NS
BPW = B // NW
PPW = BPW * NEG

NCHUNK_PAIRS = 80
NCHUNK_B = NCHUNK_PAIRS // NEG
NUM_NCHUNKS = PPW // NCHUNK_PAIRS
KC = BPW // 128


def _sc_body(ctab, xtab, cidx, xidx, nidx, pos_out, neg_out,
             cidx_v, xidx_v, nidx_v, crow_v, xrow_v, nbuf0, nbuf1,
             pos_v, negsc_v, part_v, gsem, sem0, sem1):
    w = lax.axis_index("c") * NS + lax.axis_index("s")

    pltpu.sync_copy(cidx.at[w], cidx_v)
    pltpu.sync_copy(xidx.at[w], xidx_v)
    pltpu.sync_copy(nidx.at[w], nidx_v)

    first = pltpu.async_copy(xtab.at[nidx_v.at[0]], nbuf0, sem0)

    copies = []
    for r in range(KC):
        copies.append(pltpu.async_copy(
            ctab.at[cidx_v.at[r]], crow_v.at[pl.ds(r * 128, 128)], gsem))
        copies.append(pltpu.async_copy(
            xtab.at[xidx_v.at[r]], xrow_v.at[pl.ds(r * 128, 128)], gsem))
    for cp in copies:
        cp.wait()

    nbufs = (nbuf0, nbuf1)
    sems = (sem0, sem1)
    lanes = lax.iota(jnp.int32, 16)

    def hsum16(part_ref):
        acc = plsc.load_gather(part_ref, [lanes, jnp.zeros((16,), jnp.int32)])
        for d in range(1, 16):
            acc = acc + plsc.load_gather(
                part_ref, [lanes, jnp.full((16,), d, jnp.int32)])
        return acc

    @pl.loop(0, BPW // 16)
    def _(g):
        for k in range(16):
            b = g * 16 + k
            part_v[k, :] = (
                crow_v[b, pl.ds(0, 16)] * xrow_v[b, pl.ds(0, 16)]
                + crow_v[b, pl.ds(16, 16)] * xrow_v[b, pl.ds(16, 16)]
                + crow_v[b, pl.ds(32, 16)] * xrow_v[b, pl.ds(32, 16)]
                + crow_v[b, pl.ds(48, 16)] * xrow_v[b, pl.ds(48, 16)])
        pos_v[pl.ds(g * 16, 16)] = hsum16(part_v)

    def compute_chunk(chunk, buf):
        cregs = {}
        for bb in range(NCHUNK_B):
            b = chunk * NCHUNK_B + bb
            cregs[bb] = [crow_v[b, pl.ds(16 * q, 16)] for q in range(4)]
        for grp in range(5):
            for l in range(16):
                p = grp * 16 + l
                c = cregs[p // NEG]
                part_v[l, :] = (c[0] * buf[p, pl.ds(0, 16)]
                                + c[1] * buf[p, pl.ds(16, 16)]
                                + c[2] * buf[p, pl.ds(32, 16)]
                                + c[3] * buf[p, pl.ds(48, 16)])
            negsc_v[pl.ds(chunk * NCHUNK_PAIRS + grp * 16, 16)] = hsum16(part_v)

    @pl.loop(0, NUM_NCHUNKS, step=2)
    def _(g):
        for s in range(2):
            chunk = g + s
            nxt = chunk + 1
            @pl.when(nxt < NUM_NCHUNKS)
            def _():
                pltpu.async_copy(xtab.at[nidx_v.at[nxt]],
                                 nbufs[1 - s], sems[1 - s])
            pltpu.make_async_copy(xtab.at[nidx_v.at[chunk]],
                                  nbufs[s], sems[s]).wait()
            compute_chunk(chunk, nbufs[s])

    pltpu.sync_copy(pos_v, pos_out.at[pl.ds(w * BPW, BPW)])
    pltpu.sync_copy(negsc_v, neg_out.at[pl.ds(w * PPW, PPW)])


@functools.partial(jax.jit, static_argnames=())
def _sc_scores(ctab, xtab, cidx, xidx, nidx):
    mesh = plsc.VectorSubcoreMesh(core_axis_name="c", subcore_axis_name="s")
    return pl.kernel(
        _sc_body,
        out_type=(
            jax.ShapeDtypeStruct((B,), jnp.float32),
            jax.ShapeDtypeStruct((B * NEG,), jnp.float32),
        ),
        mesh=mesh,
        scratch_types=(
            pltpu.VMEM((KC, 128), jnp.int32),
            pltpu.VMEM((KC, 128), jnp.int32),
            pltpu.VMEM((NUM_NCHUNKS, NCHUNK_PAIRS), jnp.int32),
            pltpu.VMEM((BPW, DIM), jnp.float32),
            pltpu.VMEM((BPW, DIM), jnp.float32),
            pltpu.VMEM((NCHUNK_PAIRS, DIM), jnp.float32),
            pltpu.VMEM((NCHUNK_PAIRS, DIM), jnp.float32),
            pltpu.VMEM((BPW,), jnp.float32),
            pltpu.VMEM((PPW,), jnp.float32),
            pltpu.VMEM((16, 16), jnp.float32),
            pltpu.SemaphoreType.DMA,
            pltpu.SemaphoreType.DMA,
            pltpu.SemaphoreType.DMA,
        ),
        compiler_params=pltpu.CompilerParams(
            needs_layout_passes=False, use_tc_tiling_on_sc=False),
    )(ctab, xtab, cidx, xidx, nidx)


def _tc_loss_body(pos_ref, neg_ref, out_ref):
    pos = jnp.clip(pos_ref[...], -10.0, 10.0)
    neg = jnp.clip(neg_ref[...], -10.0, 10.0)
    pos_loss = -jnp.sum(jnp.log(jax.nn.sigmoid(pos))) / B
    neg_loss = -jnp.sum(jnp.log(jax.nn.sigmoid(-neg))) / B
    out_ref[0, 0] = pos_loss + neg_loss


def _tc_loss(pos2d, neg2d):
    return pl.pallas_call(
        _tc_loss_body,
        out_shape=jax.ShapeDtypeStruct((1, 1), jnp.float32),
        out_specs=pl.BlockSpec(memory_space=pltpu.SMEM),
    )(pos2d, neg2d)


def kernel(center_word, context_word, neg_words, center_embeddings,
           context_embeddings):
    cidx = center_word.astype(jnp.int32).reshape(NW, KC, 128)
    xidx = context_word.astype(jnp.int32).reshape(NW, KC, 128)
    nidx = neg_words.astype(jnp.int32).reshape(NW, NUM_NCHUNKS, NCHUNK_PAIRS)
    pos_s, neg_s = _sc_scores(center_embeddings, context_embeddings,
                              cidx, xidx, nidx)
    loss = _tc_loss(pos_s.reshape(128, 128), neg_s.reshape(B * NEG // 128, 128))
    return loss[0, 0]
